# baseline (speedup 1.0000x reference)
import functools

import jax
import numpy as np
import jax.numpy as jnp
from jax import lax
from jax.experimental import pallas as pl
from jax.experimental.pallas import tpu as pltpu
from jax.experimental.pallas import tpu_sc as plsc

VOCAB = 100000
EMB = 256
HID = 300
BATCH = 4096
SEQ = 50

NC = 2
NS = 16
NW = NC * NS
BPW = BATCH // NW
NVB = EMB // 32
SEQP = 56
NBUF = 8


def _tree(vals):
    while len(vals) > 1:
        vals = [vals[i] + vals[i + 1] for i in range(0, len(vals) - 1, 2)] + (
            [vals[-1]] if len(vals) % 2 else [])
    return vals[0]


def _sc_gather_pool(x3d, table_i32):
    mesh = plsc.VectorSubcoreMesh(core_axis_name="c", subcore_axis_name="s")

    @functools.partial(
        pl.kernel,
        mesh=mesh,
        out_type=jax.ShapeDtypeStruct((BATCH, EMB), jnp.float32),
        scratch_types=[
            pltpu.VMEM((BPW, SEQP), jnp.int32),
        ] + [pltpu.VMEM((SEQP, EMB // 2), jnp.int32) for _ in range(NBUF)]
        + [pltpu.VMEM((BPW, EMB), jnp.float32)]
        + [pltpu.SemaphoreType.DMA for _ in range(NBUF)],
    )
    def k(x_hbm, table_hbm, out_hbm, idx_v, *rest):
        bufs = rest[:NBUF]
        outbuf = rest[NBUF]
        sems = rest[NBUF + 1:]
        t = lax.axis_index("s") * NC + lax.axis_index("c")
        pltpu.sync_copy(x_hbm.at[t], idx_v)

        def start(b, buf, sem):
            pltpu.async_copy(table_hbm.at[idx_v.at[b]], buf, sem)

        def wait(buf, sem):
            pltpu.make_async_copy(table_hbm.at[idx_v.at[0]], buf, sem).wait()

        def accum_to(buf, slot):
            C = 25
            for a in range(NVB):
                col = pl.ds(16 * a, 16)
                ev = od = None
                for r0 in range(0, SEQ, C):
                    ws = [buf[r0 + r, col] for r in range(C)]
                    e = _tree([lax.bitcast_convert_type(
                        lax.shift_left(w, 16), jnp.float32) for w in ws])
                    o = _tree([lax.bitcast_convert_type(w, jnp.float32)
                               for w in ws])
                    ev = e if ev is None else ev + e
                    od = o if od is None else od + o
                outbuf[slot, pl.ds(16 * a, 16)] = ev
                outbuf[slot, pl.ds(128 + 16 * a, 16)] = od

        for kk in range(NBUF):
            start(kk, bufs[kk], sems[kk])

        def body(i, carry):
            for kk in range(NBUF):
                e = NBUF * i + kk
                wait(bufs[kk], sems[kk])
                accum_to(bufs[kk], e)

                @pl.when(i < BPW // NBUF - 1)
                def _():
                    start(e + NBUF, bufs[kk], sems[kk])

            return carry

        lax.fori_loop(0, BPW // NBUF, body, 0)
        pltpu.sync_copy(outbuf, out_hbm.at[pl.ds(t * BPW, BPW)])

    return k(x3d, table_i32)


RPACK = 2000


def _pack_body(t_ref, o_ref):
    bits = lax.bitcast_convert_type(t_ref[...], jnp.uint32)
    ia = lax.shift_right_logical(bits[:, : EMB // 2], jnp.uint32(16))
    ib = lax.bitwise_and(bits[:, EMB // 2:], jnp.uint32(0xFFFF0000))
    o_ref[...] = lax.bitcast_convert_type(
        lax.bitwise_or(ia, ib), jnp.int32)


def _tc_pack(table):
    return pl.pallas_call(
        _pack_body,
        grid=(VOCAB // RPACK,),
        in_specs=[pl.BlockSpec((RPACK, EMB), lambda i: (i, 0))],
        out_specs=pl.BlockSpec((RPACK, EMB // 2), lambda i: (i, 0)),
        out_shape=jax.ShapeDtypeStruct((VOCAB, EMB // 2), jnp.int32),
    )(table)


BM = 512


def _mlp_body(p_ref, w_ref, b_ref, o_ref):
    p = p_ref[...].astype(jnp.float32)
    acc = lax.dot_general(
        p, w_ref[...], (((1,), (1,)), ((), ())),
        preferred_element_type=jnp.float32,
    )
    o_ref[...] = jnp.maximum(acc * (1.0 / SEQ) + b_ref[...], 0.0)


def _tc_mlp(pooled_sum, W1, b1):
    return pl.pallas_call(
        _mlp_body,
        grid=(BATCH // BM,),
        in_specs=[
            pl.BlockSpec((BM, EMB), lambda i: (i, 0)),
            pl.BlockSpec((HID, EMB), lambda i: (0, 0)),
            pl.BlockSpec((1, HID), lambda i: (0, 0)),
        ],
        out_specs=pl.BlockSpec((BM, HID), lambda i: (i, 0)),
        out_shape=jax.ShapeDtypeStruct((BATCH, HID), jnp.float32),
    )(pooled_sum, W1, b1.reshape(1, HID))


def kernel(x, table, W1, b1):
    x3d = x.astype(jnp.int32).reshape(NW, BPW, SEQ)
    x3d = jnp.concatenate([x3d, x3d[..., : SEQP - SEQ]], axis=-1)
    table_i32 = _tc_pack(table)
    pooled_sum = _sc_gather_pool(x3d, table_i32)
    return _tc_mlp(pooled_sum, W1, b1)


# speedup vs baseline: 1.0922x; 1.0922x over previous
import functools

import jax
import numpy as np
import jax.numpy as jnp
from jax import lax
from jax.experimental import pallas as pl
from jax.experimental.pallas import tpu as pltpu
from jax.experimental.pallas import tpu_sc as plsc

VOCAB = 100000
EMB = 256
HID = 300
BATCH = 4096
SEQ = 50

NC = 2
NS = 16
NW = NC * NS
BPW = BATCH // NW
NVB = EMB // 32
SEQP = 56
NBUF = 8


def _tree(vals):
    while len(vals) > 1:
        vals = [vals[i] + vals[i + 1] for i in range(0, len(vals) - 1, 2)] + (
            [vals[-1]] if len(vals) % 2 else [])
    return vals[0]


def _sc_gather_pool(x3d, table_i32):
    mesh = plsc.VectorSubcoreMesh(core_axis_name="c", subcore_axis_name="s")

    @functools.partial(
        pl.kernel,
        mesh=mesh,
        out_type=jax.ShapeDtypeStruct((BATCH, EMB), jnp.float32),
        scratch_types=[
            pltpu.VMEM((BPW, SEQP), jnp.int32),
        ] + [pltpu.VMEM((SEQP, EMB // 2), jnp.int32) for _ in range(NBUF)]
        + [pltpu.VMEM((BPW, EMB), jnp.float32)]
        + [pltpu.SemaphoreType.DMA for _ in range(NBUF)],
    )
    def k(x_hbm, table_hbm, out_hbm, idx_v, *rest):
        bufs = rest[:NBUF]
        outbuf = rest[NBUF]
        sems = rest[NBUF + 1:]
        t = lax.axis_index("s") * NC + lax.axis_index("c")
        pltpu.sync_copy(x_hbm.at[t], idx_v)

        def start(b, buf, sem):
            pltpu.async_copy(table_hbm.at[idx_v.at[b]], buf, sem)

        def wait(buf, sem):
            pltpu.make_async_copy(table_hbm.at[idx_v.at[0]], buf, sem).wait()

        def accum_to(buf, slot):
            C = 5
            for a in range(NVB):
                col = pl.ds(16 * a, 16)
                ev = od = None
                for r0 in range(0, SEQ, C):
                    ws = [buf[r0 + r, col] for r in range(C)]
                    e = _tree([lax.bitcast_convert_type(
                        lax.shift_left(w, 16), jnp.float32) for w in ws])
                    o = _tree([lax.bitcast_convert_type(w, jnp.float32)
                               for w in ws])
                    ev = e if ev is None else ev + e
                    od = o if od is None else od + o
                outbuf[slot, pl.ds(16 * a, 16)] = ev
                outbuf[slot, pl.ds(128 + 16 * a, 16)] = od

        for kk in range(NBUF):
            start(kk, bufs[kk], sems[kk])

        def body(i, carry):
            for kk in range(NBUF):
                e = NBUF * i + kk
                wait(bufs[kk], sems[kk])
                accum_to(bufs[kk], e)

                @pl.when(i < BPW // NBUF - 1)
                def _():
                    start(e + NBUF, bufs[kk], sems[kk])

            return carry

        lax.fori_loop(0, BPW // NBUF, body, 0)
        pltpu.sync_copy(outbuf, out_hbm.at[pl.ds(t * BPW, BPW)])

    return k(x3d, table_i32)


RPACK = 2000


def _pack_body(t_ref, o_ref):
    bits = lax.bitcast_convert_type(t_ref[...], jnp.uint32)
    ia = lax.shift_right_logical(bits[:, : EMB // 2], jnp.uint32(16))
    ib = lax.bitwise_and(bits[:, EMB // 2:], jnp.uint32(0xFFFF0000))
    o_ref[...] = lax.bitcast_convert_type(
        lax.bitwise_or(ia, ib), jnp.int32)


def _tc_pack(table):
    return pl.pallas_call(
        _pack_body,
        grid=(VOCAB // RPACK,),
        in_specs=[pl.BlockSpec((RPACK, EMB), lambda i: (i, 0))],
        out_specs=pl.BlockSpec((RPACK, EMB // 2), lambda i: (i, 0)),
        out_shape=jax.ShapeDtypeStruct((VOCAB, EMB // 2), jnp.int32),
    )(table)


BM = 512


def _mlp_body(p_ref, w_ref, b_ref, o_ref):
    p = p_ref[...].astype(jnp.float32)
    acc = lax.dot_general(
        p, w_ref[...], (((1,), (1,)), ((), ())),
        preferred_element_type=jnp.float32,
    )
    o_ref[...] = jnp.maximum(acc * (1.0 / SEQ) + b_ref[...], 0.0)


def _tc_mlp(pooled_sum, W1, b1):
    return pl.pallas_call(
        _mlp_body,
        grid=(BATCH // BM,),
        in_specs=[
            pl.BlockSpec((BM, EMB), lambda i: (i, 0)),
            pl.BlockSpec((HID, EMB), lambda i: (0, 0)),
            pl.BlockSpec((1, HID), lambda i: (0, 0)),
        ],
        out_specs=pl.BlockSpec((BM, HID), lambda i: (i, 0)),
        out_shape=jax.ShapeDtypeStruct((BATCH, HID), jnp.float32),
    )(pooled_sum, W1, b1.reshape(1, HID))


def kernel(x, table, W1, b1):
    x3d = x.astype(jnp.int32).reshape(NW, BPW, SEQ)
    x3d = jnp.concatenate([x3d, x3d[..., : SEQP - SEQ]], axis=-1)
    table_i32 = _tc_pack(table)
    pooled_sum = _sc_gather_pool(x3d, table_i32)
    return _tc_mlp(pooled_sum, W1, b1)


# speedup vs baseline: 1.1977x; 1.0966x over previous
import functools

import jax
import numpy as np
import jax.numpy as jnp
from jax import lax
from jax.experimental import pallas as pl
from jax.experimental.pallas import tpu as pltpu
from jax.experimental.pallas import tpu_sc as plsc

VOCAB = 100000
EMB = 256
HID = 300
BATCH = 4096
SEQ = 50

NC = 2
NS = 16
NW = NC * NS
BPW = BATCH // NW
NVB = EMB // 32
SEQP = 56
NBUF = 8


def _tree(vals):
    while len(vals) > 1:
        vals = [vals[i] + vals[i + 1] for i in range(0, len(vals) - 1, 2)] + (
            [vals[-1]] if len(vals) % 2 else [])
    return vals[0]


def _sc_gather_pool(x3d, table_i32):
    mesh = plsc.VectorSubcoreMesh(core_axis_name="c", subcore_axis_name="s")

    @functools.partial(
        pl.kernel,
        mesh=mesh,
        out_type=jax.ShapeDtypeStruct((BATCH, EMB), jnp.float32),
        scratch_types=[
            pltpu.VMEM((BPW, SEQP), jnp.int32),
        ] + [pltpu.VMEM((SEQP, EMB // 2), jnp.int32) for _ in range(NBUF)]
        + [pltpu.VMEM((BPW, EMB), jnp.float32)]
        + [pltpu.SemaphoreType.DMA for _ in range(NBUF)],
    )
    def k(x_hbm, table_hbm, out_hbm, idx_v, *rest):
        bufs = rest[:NBUF]
        outbuf = rest[NBUF]
        sems = rest[NBUF + 1:]
        t = lax.axis_index("s") * NC + lax.axis_index("c")
        pltpu.sync_copy(x_hbm.at[t], idx_v)

        def start(b, buf, sem):
            pltpu.async_copy(table_hbm.at[idx_v.at[b]], buf, sem)

        def wait(buf, sem):
            pltpu.make_async_copy(table_hbm.at[idx_v.at[0]], buf, sem).wait()

        def accum_to(buf, slot):
            C = 2
            for a in range(NVB):
                col = pl.ds(16 * a, 16)
                ev = od = None
                for r0 in range(0, SEQ, C):
                    ws = [buf[r0 + r, col] for r in range(C)]
                    e = _tree([lax.bitcast_convert_type(
                        lax.shift_left(w, 16), jnp.float32) for w in ws])
                    o = _tree([lax.bitcast_convert_type(w, jnp.float32)
                               for w in ws])
                    ev = e if ev is None else ev + e
                    od = o if od is None else od + o
                outbuf[slot, pl.ds(16 * a, 16)] = ev
                outbuf[slot, pl.ds(128 + 16 * a, 16)] = od

        for kk in range(NBUF):
            start(kk, bufs[kk], sems[kk])

        def body(i, carry):
            for kk in range(NBUF):
                e = NBUF * i + kk
                wait(bufs[kk], sems[kk])
                accum_to(bufs[kk], e)

                @pl.when(i < BPW // NBUF - 1)
                def _():
                    start(e + NBUF, bufs[kk], sems[kk])

            return carry

        lax.fori_loop(0, BPW // NBUF, body, 0)
        pltpu.sync_copy(outbuf, out_hbm.at[pl.ds(t * BPW, BPW)])

    return k(x3d, table_i32)


RPACK = 2000


def _pack_body(t_ref, o_ref):
    bits = lax.bitcast_convert_type(t_ref[...], jnp.uint32)
    ia = lax.shift_right_logical(bits[:, : EMB // 2], jnp.uint32(16))
    ib = lax.bitwise_and(bits[:, EMB // 2:], jnp.uint32(0xFFFF0000))
    o_ref[...] = lax.bitcast_convert_type(
        lax.bitwise_or(ia, ib), jnp.int32)


def _tc_pack(table):
    return pl.pallas_call(
        _pack_body,
        grid=(VOCAB // RPACK,),
        in_specs=[pl.BlockSpec((RPACK, EMB), lambda i: (i, 0))],
        out_specs=pl.BlockSpec((RPACK, EMB // 2), lambda i: (i, 0)),
        out_shape=jax.ShapeDtypeStruct((VOCAB, EMB // 2), jnp.int32),
    )(table)


BM = 512


def _mlp_body(p_ref, w_ref, b_ref, o_ref):
    p = p_ref[...].astype(jnp.float32)
    acc = lax.dot_general(
        p, w_ref[...], (((1,), (1,)), ((), ())),
        preferred_element_type=jnp.float32,
    )
    o_ref[...] = jnp.maximum(acc * (1.0 / SEQ) + b_ref[...], 0.0)


def _tc_mlp(pooled_sum, W1, b1):
    return pl.pallas_call(
        _mlp_body,
        grid=(BATCH // BM,),
        in_specs=[
            pl.BlockSpec((BM, EMB), lambda i: (i, 0)),
            pl.BlockSpec((HID, EMB), lambda i: (0, 0)),
            pl.BlockSpec((1, HID), lambda i: (0, 0)),
        ],
        out_specs=pl.BlockSpec((BM, HID), lambda i: (i, 0)),
        out_shape=jax.ShapeDtypeStruct((BATCH, HID), jnp.float32),
    )(pooled_sum, W1, b1.reshape(1, HID))


def kernel(x, table, W1, b1):
    x3d = x.astype(jnp.int32).reshape(NW, BPW, SEQ)
    x3d = jnp.concatenate([x3d, x3d[..., : SEQP - SEQ]], axis=-1)
    table_i32 = _tc_pack(table)
    pooled_sum = _sc_gather_pool(x3d, table_i32)
    return _tc_mlp(pooled_sum, W1, b1)
